# Initial kernel scaffold; baseline (speedup 1.0000x reference)
import functools

import jax
import jax.numpy as jnp
from jax import lax
from jax.experimental import pallas as pl
from jax.experimental.pallas import tpu as pltpu
from jax.experimental.pallas import tpu_sc as plsc

NC = 2
NS = 16
NW = NC * NS

EDGE_CH = 128


def _sc_mesh():
    return plsc.VectorSubcoreMesh(core_axis_name="c", subcore_axis_name="s")


def _sc_degree(dst3, zeros_rows, ones_vec, acc_rows, rows_per_tile):
    cpt = dst3.shape[1]

    @functools.partial(
        pl.kernel,
        mesh=_sc_mesh(),
        out_type=jax.ShapeDtypeStruct((NC, acc_rows), jnp.float32),
        scratch_types=[
            pltpu.VMEM((cpt, EDGE_CH), jnp.int32),
            pltpu.VMEM((EDGE_CH,), jnp.float32),
            pltpu.VMEM_SHARED((acc_rows,), jnp.float32),
        ],
    )
    def deg_kernel(dst_hbm, z_hbm, ones_hbm, out_hbm, didx, ones_v, acc):
        c = lax.axis_index("c")
        s = lax.axis_index("s")
        wid = c * NS + s
        base = s * rows_per_tile
        pltpu.sync_copy(z_hbm, acc.at[pl.ds(base, rows_per_tile)])
        pltpu.sync_copy(ones_hbm, ones_v)
        pltpu.sync_copy(dst_hbm.at[wid], didx)
        plsc.subcore_barrier()

        @pl.loop(0, cpt)
        def _(j):
            pltpu.sync_copy(ones_v, acc.at[didx.at[j]], add=True)

        plsc.subcore_barrier()
        pltpu.sync_copy(acc.at[pl.ds(base, rows_per_tile)],
                        out_hbm.at[c, pl.ds(base, rows_per_tile)])

    return deg_kernel(dst3, zeros_rows, ones_vec)


def _sc_scatter_layer(g, src3, dst3, zrows, acc_rows, rows_per_tile):
    d = g.shape[1]
    cpt = src3.shape[1]

    @functools.partial(
        pl.kernel,
        mesh=_sc_mesh(),
        out_type=jax.ShapeDtypeStruct((NC, acc_rows, d), jnp.float32),
        scratch_types=[
            pltpu.VMEM((cpt, EDGE_CH), jnp.int32),
            pltpu.VMEM((cpt, EDGE_CH), jnp.int32),
            pltpu.VMEM((2, EDGE_CH, d), jnp.float32),
            pltpu.VMEM_SHARED((acc_rows, d), jnp.float32),
            pltpu.SemaphoreType.DMA,
            pltpu.SemaphoreType.DMA,
        ],
    )
    def layer_kernel(g_hbm, src_hbm, dst_hbm, z_hbm, out_hbm,
                     sidx, didx, rows, acc, sem0, sem1):
        c = lax.axis_index("c")
        s = lax.axis_index("s")
        wid = c * NS + s
        base = s * rows_per_tile
        pltpu.sync_copy(z_hbm, acc.at[pl.ds(base, rows_per_tile)])
        pltpu.sync_copy(src_hbm.at[wid], sidx)
        pltpu.sync_copy(dst_hbm.at[wid], didx)
        plsc.subcore_barrier()

        pltpu.async_copy(g_hbm.at[sidx.at[0]], rows.at[0], sem0)

        @pl.loop(0, cpt, step=2)
        def _(j):
            @pl.when(j + 1 < cpt)
            def _():
                pltpu.async_copy(g_hbm.at[sidx.at[j + 1]], rows.at[1], sem1)

            pltpu.make_async_copy(g_hbm.at[sidx.at[j]], rows.at[0], sem0).wait()
            pltpu.sync_copy(rows.at[0], acc.at[didx.at[j]], add=True)

            @pl.when(j + 2 < cpt)
            def _():
                pltpu.async_copy(g_hbm.at[sidx.at[j + 2]], rows.at[0], sem0)

            @pl.when(j + 1 < cpt)
            def _():
                pltpu.make_async_copy(g_hbm.at[sidx.at[j + 1]], rows.at[1],
                                      sem1).wait()
                pltpu.sync_copy(rows.at[1], acc.at[didx.at[j + 1]], add=True)

        plsc.subcore_barrier()
        pltpu.sync_copy(acc.at[pl.ds(base, rows_per_tile)],
                        out_hbm.at[c, pl.ds(base, rows_per_tile)])

    return layer_kernel(g, src3, dst3, zrows)


def _tc_first(x, W, d0, d1, blk):
    n, d = x.shape
    grid = n // blk

    def body(x_ref, w_ref, d0_ref, d1_ref, g_ref, dinv_ref):
        deg = d0_ref[...] + d1_ref[...] + 1.0
        dinv = lax.rsqrt(deg)
        dinv_ref[...] = dinv
        z = jnp.dot(x_ref[...], w_ref[...], preferred_element_type=jnp.float32)
        g_ref[...] = z * dinv

    return pl.pallas_call(
        body,
        grid=(grid,),
        in_specs=[
            pl.BlockSpec((blk, d), lambda i: (i, 0)),
            pl.BlockSpec((d, d), lambda i: (0, 0)),
            pl.BlockSpec((blk, 1), lambda i: (i, 0)),
            pl.BlockSpec((blk, 1), lambda i: (i, 0)),
        ],
        out_specs=[
            pl.BlockSpec((blk, d), lambda i: (i, 0)),
            pl.BlockSpec((blk, 1), lambda i: (i, 0)),
        ],
        out_shape=[
            jax.ShapeDtypeStruct((n, d), jnp.float32),
            jax.ShapeDtypeStruct((n, 1), jnp.float32),
        ],
    )(x, W, d0, d1)


def _tc_mid(p0, p1, g1, dinv, W, b2, blk):
    n, d = g1.shape
    grid = n // blk

    def body(p0_ref, p1_ref, g_ref, dinv_ref, w_ref, b_ref, out_ref):
        dinv = dinv_ref[...]
        h = (p0_ref[...] + p1_ref[...] + g_ref[...]) * dinv + b_ref[...]
        h = jnp.maximum(h, 0.0)
        z = jnp.dot(h, w_ref[...], preferred_element_type=jnp.float32)
        out_ref[...] = z * dinv

    return pl.pallas_call(
        body,
        grid=(grid,),
        in_specs=[
            pl.BlockSpec((blk, d), lambda i: (i, 0)),
            pl.BlockSpec((blk, d), lambda i: (i, 0)),
            pl.BlockSpec((blk, d), lambda i: (i, 0)),
            pl.BlockSpec((blk, 1), lambda i: (i, 0)),
            pl.BlockSpec((d, d), lambda i: (0, 0)),
            pl.BlockSpec((1, d), lambda i: (0, 0)),
        ],
        out_specs=pl.BlockSpec((blk, d), lambda i: (i, 0)),
        out_shape=jax.ShapeDtypeStruct((n, d), jnp.float32),
    )(p0, p1, g1, dinv, W, b2)


def _tc_last(p0, p1, g2, dinv, b2, blk):
    n, d = g2.shape
    grid = n // blk

    def body(p0_ref, p1_ref, g_ref, dinv_ref, b_ref, out_ref):
        h = (p0_ref[...] + p1_ref[...] + g_ref[...]) * dinv_ref[...] + b_ref[...]
        out_ref[...] = jnp.maximum(h, 0.0)

    return pl.pallas_call(
        body,
        grid=(grid,),
        in_specs=[
            pl.BlockSpec((blk, d), lambda i: (i, 0)),
            pl.BlockSpec((blk, d), lambda i: (i, 0)),
            pl.BlockSpec((blk, d), lambda i: (i, 0)),
            pl.BlockSpec((blk, 1), lambda i: (i, 0)),
            pl.BlockSpec((1, d), lambda i: (0, 0)),
        ],
        out_specs=pl.BlockSpec((blk, d), lambda i: (i, 0)),
        out_shape=jax.ShapeDtypeStruct((n, d), jnp.float32),
    )(p0, p1, g2, dinv, b2)


def kernel(x, edge_index, batch_index, node_rankings, W, b):
    n, d = x.shape
    e = edge_index.shape[1]

    cpt = -(-e // (NW * EDGE_CH))
    e_pad = NW * cpt * EDGE_CH
    rows_per_tile = -(-(-(-n // NS)) // 8) * 8
    acc_rows = NS * rows_per_tile
    n_dummy = acc_rows - n

    src = edge_index[0].astype(jnp.int32)
    dst = edge_index[1].astype(jnp.int32)
    npad = e_pad - e
    pad_i = jnp.arange(npad, dtype=jnp.int32)
    pad_src = (pad_i * 61) % n
    pad_dst = n + pad_i % n_dummy
    src3 = jnp.concatenate([src, pad_src]).reshape(NW, cpt, EDGE_CH)
    dst3 = jnp.concatenate([dst, pad_dst]).reshape(NW, cpt, EDGE_CH)

    zrow1 = jnp.zeros((rows_per_tile,), jnp.float32)
    zrows = jnp.zeros((rows_per_tile, d), jnp.float32)
    ones_vec = jnp.ones((EDGE_CH,), jnp.float32)

    degp = _sc_degree(dst3, zrow1, ones_vec, acc_rows, rows_per_tile)
    d0 = degp[0, :n].reshape(n, 1)
    d1 = degp[1, :n].reshape(n, 1)

    blk = 1000
    g1, dinv = _tc_first(x, W, d0, d1, blk)

    b2 = b.reshape(1, d)
    p = _sc_scatter_layer(g1, src3, dst3, zrows, acc_rows, rows_per_tile)
    g2 = _tc_mid(p[0, :n], p[1, :n], g1, dinv, W, b2, blk)
    p2 = _sc_scatter_layer(g2, src3, dst3, zrows, acc_rows, rows_per_tile)
    return _tc_last(p2[0, :n], p2[1, :n], g2, dinv, b2, blk)


# speedup vs baseline: 28.6920x; 28.6920x over previous
import functools

import jax
import jax.numpy as jnp
from jax import lax
from jax.experimental import pallas as pl
from jax.experimental.pallas import tpu as pltpu
from jax.experimental.pallas import tpu_sc as plsc

NC = 2
NS = 16
NW = NC * NS

EDGE_CH = 128
CPG = 8


def _sc_mesh():
    return plsc.VectorSubcoreMesh(core_axis_name="c", subcore_axis_name="s")


def _sc_degree(dst3, zeros_rows, ones_vec, acc_rows, rows_per_tile):
    cpt = dst3.shape[1]

    @functools.partial(
        pl.kernel,
        mesh=_sc_mesh(),
        out_type=jax.ShapeDtypeStruct((NC * acc_rows,), jnp.float32),
        scratch_types=[
            pltpu.VMEM((cpt, EDGE_CH), jnp.int32),
            pltpu.VMEM((EDGE_CH,), jnp.float32),
            pltpu.VMEM((rows_per_tile,), jnp.float32),
            pltpu.VMEM_SHARED((acc_rows,), jnp.float32),
        ],
    )
    def deg_kernel(dst_hbm, z_hbm, ones_hbm, out_hbm, didx, ones_v, vbuf, acc):
        c = lax.axis_index("c")
        s = lax.axis_index("s")
        wid = c * NS + s
        base = s * rows_per_tile
        pltpu.sync_copy(z_hbm, vbuf)
        pltpu.sync_copy(vbuf, acc.at[pl.ds(base, rows_per_tile)])
        pltpu.sync_copy(ones_hbm, ones_v)
        pltpu.sync_copy(dst_hbm.at[wid], didx)
        plsc.subcore_barrier()

        @pl.loop(0, cpt)
        def _(j):
            pltpu.sync_copy(ones_v, acc.at[didx.at[j]], add=True)

        plsc.subcore_barrier()
        pltpu.sync_copy(acc.at[pl.ds(base, rows_per_tile)], vbuf)
        pltpu.sync_copy(vbuf,
                        out_hbm.at[pl.ds(c * acc_rows + base, rows_per_tile)])

    return deg_kernel(dst3, zeros_rows, ones_vec).reshape(NC, acc_rows)


def _sc_scatter_layer(g, src3, dst3, zrows, acc_rows, rows_per_tile):
    d = g.shape[1]
    cpt = src3.shape[1]
    groups = cpt // CPG
    nfull = rows_per_tile // EDGE_CH
    rem = rows_per_tile % EDGE_CH

    @functools.partial(
        pl.kernel,
        mesh=_sc_mesh(),
        out_type=jax.ShapeDtypeStruct((NC, acc_rows, d), jnp.float32),
        scratch_types=[
            pltpu.VMEM((2, CPG, EDGE_CH), jnp.int32),
            pltpu.VMEM((2, CPG, EDGE_CH), jnp.int32),
            pltpu.VMEM((2, EDGE_CH, d), jnp.float32),
            pltpu.VMEM_SHARED((acc_rows, d), jnp.float32),
            pltpu.SemaphoreType.DMA,
            pltpu.SemaphoreType.DMA,
            pltpu.SemaphoreType.DMA,
            pltpu.SemaphoreType.DMA,
        ],
    )
    def layer_kernel(g_hbm, src_hbm, dst_hbm, z_hbm, out_hbm,
                     sidx, didx, rows, acc, sem0, sem1, sem_si, sem_di):
        c = lax.axis_index("c")
        s = lax.axis_index("s")
        wid = c * NS + s
        base = s * rows_per_tile
        pltpu.sync_copy(z_hbm, rows.at[0])
        for k in range(nfull):
            pltpu.sync_copy(rows.at[0],
                            acc.at[pl.ds(base + k * EDGE_CH, EDGE_CH)])
        if rem:
            pltpu.sync_copy(rows.at[0, pl.ds(0, rem)],
                            acc.at[pl.ds(base + nfull * EDGE_CH, rem)])
        pltpu.sync_copy(src_hbm.at[wid, pl.ds(0, CPG)], sidx.at[0])
        pltpu.sync_copy(dst_hbm.at[wid, pl.ds(0, CPG)], didx.at[0])
        plsc.subcore_barrier()

        pltpu.async_copy(g_hbm.at[sidx.at[0, 0]], rows.at[0], sem0)
        sems = (sem0, sem1)

        @pl.loop(0, groups)
        def _(gi):
            gb = lax.rem(gi, 2)
            @pl.when(gi + 1 < groups)
            def _():
                pltpu.async_copy(
                    src_hbm.at[wid, pl.ds((gi + 1) * CPG, CPG)],
                    sidx.at[1 - gb], sem_si)
                pltpu.async_copy(
                    dst_hbm.at[wid, pl.ds((gi + 1) * CPG, CPG)],
                    didx.at[1 - gb], sem_di)

            for jj in range(CPG):
                p = jj % 2
                if jj + 1 < CPG:
                    pltpu.async_copy(g_hbm.at[sidx.at[gb, jj + 1]],
                                     rows.at[1 - p], sems[1 - p])
                pltpu.make_async_copy(g_hbm.at[sidx.at[gb, jj]],
                                      rows.at[p], sems[p]).wait()
                pltpu.sync_copy(rows.at[p], acc.at[didx.at[gb, jj]], add=True)
                if jj == CPG - 1:
                    @pl.when(gi + 1 < groups)
                    def _():
                        pltpu.make_async_copy(
                            src_hbm.at[wid, pl.ds((gi + 1) * CPG, CPG)],
                            sidx.at[1 - gb], sem_si).wait()
                        pltpu.make_async_copy(
                            dst_hbm.at[wid, pl.ds((gi + 1) * CPG, CPG)],
                            didx.at[1 - gb], sem_di).wait()
                        pltpu.async_copy(g_hbm.at[sidx.at[1 - gb, 0]],
                                         rows.at[0], sem0)

        plsc.subcore_barrier()
        for k in range(nfull):
            buf = rows.at[k % 2]
            pltpu.sync_copy(acc.at[pl.ds(base + k * EDGE_CH, EDGE_CH)], buf)
            pltpu.sync_copy(buf,
                            out_hbm.at[c, pl.ds(base + k * EDGE_CH, EDGE_CH)])
        if rem:
            buf = rows.at[nfull % 2, pl.ds(0, rem)]
            pltpu.sync_copy(acc.at[pl.ds(base + nfull * EDGE_CH, rem)], buf)
            pltpu.sync_copy(
                buf, out_hbm.at[c, pl.ds(base + nfull * EDGE_CH, rem)])

    return layer_kernel(g, src3, dst3, zrows)


def _tc_first(x, W, d0, d1, blk):
    n, d = x.shape
    grid = n // blk

    def body(x_ref, w_ref, d0_ref, d1_ref, g_ref, dinv_ref):
        deg = d0_ref[...] + d1_ref[...] + 1.0
        dinv = lax.rsqrt(deg)
        dinv_ref[...] = dinv
        z = jnp.dot(x_ref[...], w_ref[...], preferred_element_type=jnp.float32)
        g_ref[...] = z * dinv

    return pl.pallas_call(
        body,
        grid=(grid,),
        in_specs=[
            pl.BlockSpec((blk, d), lambda i: (i, 0)),
            pl.BlockSpec((d, d), lambda i: (0, 0)),
            pl.BlockSpec((blk, 1), lambda i: (i, 0)),
            pl.BlockSpec((blk, 1), lambda i: (i, 0)),
        ],
        out_specs=[
            pl.BlockSpec((blk, d), lambda i: (i, 0)),
            pl.BlockSpec((blk, 1), lambda i: (i, 0)),
        ],
        out_shape=[
            jax.ShapeDtypeStruct((n, d), jnp.float32),
            jax.ShapeDtypeStruct((n, 1), jnp.float32),
        ],
    )(x, W, d0, d1)


def _tc_mid(p0, p1, g1, dinv, W, b2, blk):
    n, d = g1.shape
    grid = n // blk

    def body(p0_ref, p1_ref, g_ref, dinv_ref, w_ref, b_ref, out_ref):
        dinv = dinv_ref[...]
        h = (p0_ref[...] + p1_ref[...] + g_ref[...]) * dinv + b_ref[...]
        h = jnp.maximum(h, 0.0)
        z = jnp.dot(h, w_ref[...], preferred_element_type=jnp.float32)
        out_ref[...] = z * dinv

    return pl.pallas_call(
        body,
        grid=(grid,),
        in_specs=[
            pl.BlockSpec((blk, d), lambda i: (i, 0)),
            pl.BlockSpec((blk, d), lambda i: (i, 0)),
            pl.BlockSpec((blk, d), lambda i: (i, 0)),
            pl.BlockSpec((blk, 1), lambda i: (i, 0)),
            pl.BlockSpec((d, d), lambda i: (0, 0)),
            pl.BlockSpec((1, d), lambda i: (0, 0)),
        ],
        out_specs=pl.BlockSpec((blk, d), lambda i: (i, 0)),
        out_shape=jax.ShapeDtypeStruct((n, d), jnp.float32),
    )(p0, p1, g1, dinv, W, b2)


def _tc_last(p0, p1, g2, dinv, b2, blk):
    n, d = g2.shape
    grid = n // blk

    def body(p0_ref, p1_ref, g_ref, dinv_ref, b_ref, out_ref):
        h = (p0_ref[...] + p1_ref[...] + g_ref[...]) * dinv_ref[...] + b_ref[...]
        out_ref[...] = jnp.maximum(h, 0.0)

    return pl.pallas_call(
        body,
        grid=(grid,),
        in_specs=[
            pl.BlockSpec((blk, d), lambda i: (i, 0)),
            pl.BlockSpec((blk, d), lambda i: (i, 0)),
            pl.BlockSpec((blk, d), lambda i: (i, 0)),
            pl.BlockSpec((blk, 1), lambda i: (i, 0)),
            pl.BlockSpec((1, d), lambda i: (0, 0)),
        ],
        out_specs=pl.BlockSpec((blk, d), lambda i: (i, 0)),
        out_shape=jax.ShapeDtypeStruct((n, d), jnp.float32),
    )(p0, p1, g2, dinv, b2)


def kernel(x, edge_index, batch_index, node_rankings, W, b):
    n, d = x.shape
    e = edge_index.shape[1]

    cpt = -(-e // (NW * EDGE_CH * CPG)) * CPG
    e_pad = NW * cpt * EDGE_CH
    rows_per_tile = -(-(-(-n // NS)) // 8) * 8
    acc_rows = NS * rows_per_tile
    n_dummy = acc_rows - n

    src = edge_index[0].astype(jnp.int32)
    dst = edge_index[1].astype(jnp.int32)
    npad = e_pad - e
    pad_i = jnp.arange(npad, dtype=jnp.int32)
    pad_src = (pad_i * 61) % n
    pad_dst = n + pad_i % n_dummy
    src3 = jnp.concatenate([src, pad_src]).reshape(NW, cpt, EDGE_CH)
    dst3 = jnp.concatenate([dst, pad_dst]).reshape(NW, cpt, EDGE_CH)

    zrow1 = jnp.zeros((rows_per_tile,), jnp.float32)
    zrows = jnp.zeros((EDGE_CH, d), jnp.float32)
    ones_vec = jnp.ones((EDGE_CH,), jnp.float32)

    degp = _sc_degree(dst3, zrow1, ones_vec, acc_rows, rows_per_tile)
    d0 = degp[0, :n].reshape(n, 1)
    d1 = degp[1, :n].reshape(n, 1)

    blk = 1000
    g1, dinv = _tc_first(x, W, d0, d1, blk)

    b2 = b.reshape(1, d)
    p = _sc_scatter_layer(g1, src3, dst3, zrows, acc_rows, rows_per_tile)
    g2 = _tc_mid(p[0, :n], p[1, :n], g1, dinv, W, b2, blk)
    p2 = _sc_scatter_layer(g2, src3, dst3, zrows, acc_rows, rows_per_tile)
    return _tc_last(p2[0, :n], p2[1, :n], g2, dinv, b2, blk)
